# baseline (speedup 1.0000x reference)
import functools

import jax
import jax.numpy as jnp
from jax import lax
from jax.experimental import pallas as pl
from jax.experimental.pallas import tpu as pltpu
from jax.experimental.pallas import tpu_sc as plsc

D_MODEL = 128
T_LEN = 2048
N_OUTER = 256
T_CHUNK = 256
N_TCH = T_LEN // T_CHUNK
CHUNK = T_CHUNK * D_MODEL
N_WORKERS = 32
ROWS_PER_W = N_OUTER // N_WORKERS
LANES = 16
UNROLL = 8


def _add_inplace(dst_ref, src_ref):
    def body(i, carry):
        base = i * (LANES * UNROLL)
        for u in range(UNROLL):
            off = base + u * LANES
            dst_ref[pl.ds(off, LANES)] = (
                dst_ref[pl.ds(off, LANES)] + src_ref[pl.ds(off, LANES)]
            )
        return carry
    lax.fori_loop(0, CHUNK // (LANES * UNROLL), body, 0)


def _sc_body(x_hbm, l_hbm, f_hbm, out_hbm, xb, peb, tb):
    wid = lax.axis_index("s") * 2 + lax.axis_index("c")

    def tc_body(tc, carry):
        pltpu.sync_copy(l_hbm.at[tc], peb)
        pltpu.sync_copy(f_hbm.at[tc], tb)
        _add_inplace(peb, tb)

        def r_body(r, c2):
            row = wid * ROWS_PER_W + r
            pltpu.sync_copy(x_hbm.at[row, tc], xb)
            _add_inplace(xb, peb)
            pltpu.sync_copy(xb, out_hbm.at[row, tc])
            return c2

        lax.fori_loop(0, ROWS_PER_W, r_body, 0)
        return carry

    lax.fori_loop(0, N_TCH, tc_body, 0)


@jax.jit
def _run(x3, l2, f2):
    mesh = plsc.VectorSubcoreMesh(core_axis_name="c", subcore_axis_name="s")
    kern = functools.partial(
        pl.kernel,
        mesh=mesh,
        out_type=jax.ShapeDtypeStruct((N_OUTER, N_TCH, CHUNK), jnp.float32),
        scratch_types=[
            pltpu.VMEM((CHUNK,), jnp.float32),
            pltpu.VMEM((CHUNK,), jnp.float32),
            pltpu.VMEM((CHUNK,), jnp.float32),
        ],
    )(_sc_body)
    return kern(x3, l2, f2)


def kernel(x, learned_pe_table, fixed_pe):
    B, N, T, D = x.shape
    x3 = x.reshape(N_OUTER, N_TCH, CHUNK)
    l2 = learned_pe_table.reshape(N_TCH, CHUNK)
    f2 = fixed_pe.reshape(N_TCH, CHUNK)
    out = _run(x3, l2, f2)
    return out.reshape(B, N, T, D)


# speedup vs baseline: 1.2962x; 1.2962x over previous
import functools

import jax
import jax.numpy as jnp
from jax import lax
from jax.experimental import pallas as pl
from jax.experimental.pallas import tpu as pltpu
from jax.experimental.pallas import tpu_sc as plsc

D_MODEL = 128
T_LEN = 2048
N_OUTER = 256
T_CHUNK = 128
N_TCH = T_LEN // T_CHUNK
CHUNK = T_CHUNK * D_MODEL
ROWS_PER_W = N_OUTER // 2
LANES = 16
UNROLL = 8
NB = 2


def _sc_body(x_hbm, l_hbm, f_hbm, out_hbm,
             xb0, xb1, ob0, ob1, peb,
             in_sem0, in_sem1, out_sem0, out_sem1):
    c = lax.axis_index("c")
    s = lax.axis_index("s")
    tch = s
    row0 = c * ROWS_PER_W
    xbufs = [xb0, xb1]
    obufs = [ob0, ob1]
    in_sems = [in_sem0, in_sem1]
    out_sems = [out_sem0, out_sem1]

    pltpu.sync_copy(l_hbm.at[tch], peb)
    pltpu.sync_copy(f_hbm.at[tch], ob0)

    def pe_body(i, carry):
        base = i * (LANES * UNROLL)
        for u in range(UNROLL):
            off = base + u * LANES
            peb[pl.ds(off, LANES)] = (
                peb[pl.ds(off, LANES)] + ob0[pl.ds(off, LANES)]
            )
        return carry
    lax.fori_loop(0, CHUNK // (LANES * UNROLL), pe_body, 0)

    for b in range(NB):
        pltpu.make_async_copy(x_hbm.at[row0 + b, tch], xbufs[b], in_sems[b]).start()

    def add_chunk(dst, src):
        def body(i, carry):
            base = i * (LANES * UNROLL)
            for u in range(UNROLL):
                off = base + u * LANES
                dst[pl.ds(off, LANES)] = (
                    src[pl.ds(off, LANES)] + peb[pl.ds(off, LANES)]
                )
            return carry
        lax.fori_loop(0, CHUNK // (LANES * UNROLL), body, 0)

    def g_body(g, carry):
        for b in range(NB):
            r = g * NB + b
            row = row0 + r
            pltpu.make_async_copy(x_hbm.at[row, tch], xbufs[b],
                                  in_sems[b]).wait()

            @pl.when(g > 0)
            def _():
                pltpu.make_async_copy(
                    obufs[b], out_hbm.at[row - NB, tch], out_sems[b]).wait()

            add_chunk(obufs[b], xbufs[b])
            pltpu.make_async_copy(obufs[b], out_hbm.at[row, tch],
                                  out_sems[b]).start()

            @pl.when(r + NB < ROWS_PER_W)
            def _():
                pltpu.make_async_copy(x_hbm.at[row + NB, tch], xbufs[b],
                                      in_sems[b]).start()
        return carry

    lax.fori_loop(0, ROWS_PER_W // NB, g_body, 0)

    for b in range(NB):
        row = row0 + ROWS_PER_W - NB + b
        pltpu.make_async_copy(obufs[b], out_hbm.at[row, tch],
                              out_sems[b]).wait()


@jax.jit
def _run(x3, l2, f2):
    mesh = plsc.VectorSubcoreMesh(core_axis_name="c", subcore_axis_name="s")
    kern = functools.partial(
        pl.kernel,
        mesh=mesh,
        out_type=jax.ShapeDtypeStruct((N_OUTER, N_TCH, CHUNK), jnp.float32),
        scratch_types=[
            pltpu.VMEM((CHUNK,), jnp.float32),
            pltpu.VMEM((CHUNK,), jnp.float32),
            pltpu.VMEM((CHUNK,), jnp.float32),
            pltpu.VMEM((CHUNK,), jnp.float32),
            pltpu.VMEM((CHUNK,), jnp.float32),
            pltpu.SemaphoreType.DMA,
            pltpu.SemaphoreType.DMA,
            pltpu.SemaphoreType.DMA,
            pltpu.SemaphoreType.DMA,
        ],
    )(_sc_body)
    return kern(x3, l2, f2)


def kernel(x, learned_pe_table, fixed_pe):
    B, N, T, D = x.shape
    x3 = x.reshape(N_OUTER, N_TCH, CHUNK)
    l2 = learned_pe_table.reshape(N_TCH, CHUNK)
    f2 = fixed_pe.reshape(N_TCH, CHUNK)
    out = _run(x3, l2, f2)
    return out.reshape(B, N, T, D)


# speedup vs baseline: 1.3260x; 1.0229x over previous
import functools

import jax
import jax.numpy as jnp
from jax import lax
from jax.experimental import pallas as pl
from jax.experimental.pallas import tpu as pltpu
from jax.experimental.pallas import tpu_sc as plsc

D_MODEL = 128
T_LEN = 2048
N_OUTER = 256
T_CHUNK = 64
N_TCH = T_LEN // T_CHUNK
CHUNK = T_CHUNK * D_MODEL
LANES = 16
UNROLL = 8
NB = 4


def _sc_body(x_hbm, l_hbm, f_hbm, out_hbm,
             xb0, xb1, xb2, xb3, ob0, ob1, ob2, ob3, peb,
             isem0, isem1, isem2, isem3, osem0, osem1, osem2, osem3):
    c = lax.axis_index("c")
    s = lax.axis_index("s")
    tch = s * 2 + c
    xbufs = [xb0, xb1, xb2, xb3]
    obufs = [ob0, ob1, ob2, ob3]
    in_sems = [isem0, isem1, isem2, isem3]
    out_sems = [osem0, osem1, osem2, osem3]

    pltpu.sync_copy(l_hbm.at[tch], peb)
    pltpu.sync_copy(f_hbm.at[tch], ob0)

    def pe_body(i, carry):
        base = i * (LANES * UNROLL)
        for u in range(UNROLL):
            off = base + u * LANES
            peb[pl.ds(off, LANES)] = (
                peb[pl.ds(off, LANES)] + ob0[pl.ds(off, LANES)]
            )
        return carry
    lax.fori_loop(0, CHUNK // (LANES * UNROLL), pe_body, 0)

    for b in range(NB):
        pltpu.make_async_copy(x_hbm.at[b, tch], xbufs[b], in_sems[b]).start()

    def add_chunk(dst, src):
        def body(i, carry):
            base = i * (LANES * UNROLL)
            for u in range(UNROLL):
                off = base + u * LANES
                dst[pl.ds(off, LANES)] = (
                    src[pl.ds(off, LANES)] + peb[pl.ds(off, LANES)]
                )
            return carry
        lax.fori_loop(0, CHUNK // (LANES * UNROLL), body, 0)

    def g_body(g, carry):
        for b in range(NB):
            row = g * NB + b
            pltpu.make_async_copy(x_hbm.at[row, tch], xbufs[b],
                                  in_sems[b]).wait()

            @pl.when(g > 0)
            def _():
                pltpu.make_async_copy(
                    obufs[b], out_hbm.at[row - NB, tch], out_sems[b]).wait()

            add_chunk(obufs[b], xbufs[b])
            pltpu.make_async_copy(obufs[b], out_hbm.at[row, tch],
                                  out_sems[b]).start()

            @pl.when(row + NB < N_OUTER)
            def _():
                pltpu.make_async_copy(x_hbm.at[row + NB, tch], xbufs[b],
                                      in_sems[b]).start()
        return carry

    lax.fori_loop(0, N_OUTER // NB, g_body, 0)

    for b in range(NB):
        row = N_OUTER - NB + b
        pltpu.make_async_copy(obufs[b], out_hbm.at[row, tch],
                              out_sems[b]).wait()


@jax.jit
def _run(x3, l2, f2):
    mesh = plsc.VectorSubcoreMesh(core_axis_name="c", subcore_axis_name="s")
    kern = functools.partial(
        pl.kernel,
        mesh=mesh,
        out_type=jax.ShapeDtypeStruct((N_OUTER, N_TCH, CHUNK), jnp.float32),
        scratch_types=(
            [pltpu.VMEM((CHUNK,), jnp.float32)] * 9
            + [pltpu.SemaphoreType.DMA] * 8
        ),
    )(_sc_body)
    return kern(x3, l2, f2)


def kernel(x, learned_pe_table, fixed_pe):
    B, N, T, D = x.shape
    x3 = x.reshape(N_OUTER, N_TCH, CHUNK)
    l2 = learned_pe_table.reshape(N_TCH, CHUNK)
    f2 = fixed_pe.reshape(N_TCH, CHUNK)
    out = _run(x3, l2, f2)
    return out.reshape(B, N, T, D)


# speedup vs baseline: 6.2735x; 4.7313x over previous
import functools

import jax
import jax.numpy as jnp
from jax.experimental import pallas as pl
from jax.experimental.pallas import tpu as pltpu

D_MODEL = 128
T_LEN = 2048
N_OUTER = 256
R_BLK = 8


def _tc_body(x_ref, l_ref, f_ref, o_ref):
    o_ref[...] = x_ref[...] + (l_ref[...] + f_ref[...])[None]


@jax.jit
def _run(x3, l2, f2):
    grid = (N_OUTER // R_BLK,)
    return pl.pallas_call(
        _tc_body,
        grid=grid,
        in_specs=[
            pl.BlockSpec((R_BLK, T_LEN, D_MODEL), lambda i: (i, 0, 0)),
            pl.BlockSpec((T_LEN, D_MODEL), lambda i: (0, 0)),
            pl.BlockSpec((T_LEN, D_MODEL), lambda i: (0, 0)),
        ],
        out_specs=pl.BlockSpec((R_BLK, T_LEN, D_MODEL), lambda i: (i, 0, 0)),
        out_shape=jax.ShapeDtypeStruct((N_OUTER, T_LEN, D_MODEL), jnp.float32),
    )(x3, l2, f2)


def kernel(x, learned_pe_table, fixed_pe):
    B, N, T, D = x.shape
    x3 = x.reshape(N_OUTER, T_LEN, D_MODEL)
    l2 = learned_pe_table
    f2 = fixed_pe.reshape(T_LEN, D_MODEL)
    out = _run(x3, l2, f2)
    return out.reshape(B, N, T, D)
